# Initial kernel scaffold; baseline (speedup 1.0000x reference)
import jax
import jax.numpy as jnp
from jax.experimental import pallas as pl


def kernel(input_features, neighbor_inds, inverse_neighbors, inverse_k, inverse_idx, weightnet, linear_weight, linear_bias):
    raise NotImplementedError("write your pallas kernel here")


# speedup vs baseline: 2.8728x; 2.8728x over previous
import functools

import jax
import jax.numpy as jnp
from jax import lax
from jax.experimental import pallas as pl
from jax.experimental.pallas import tpu as pltpu
from jax.experimental.pallas import tpu_sc as plsc

_B, _N, _K, _C, _M, _OUT = 1, 100000, 16, 16, 16, 64


_NW = 32
_PER_W = (_N * _K) // _NW
_CH = 2000
_NCH = _PER_W // _CH


def _sc_gather_body(idx_hbm, table_hbm, out_hbm, idx_v, rows_v, sem):
    wid = lax.axis_index("s") * 2 + lax.axis_index("c")

    def chunk(j, carry):
        base = wid * _PER_W + j * _CH
        pltpu.sync_copy(idx_hbm.at[pl.ds(base, _CH)], idx_v)
        pltpu.async_copy(table_hbm.at[idx_v], rows_v, sem).wait()
        pltpu.sync_copy(rows_v, out_hbm.at[pl.ds(base, _CH)])
        return carry

    lax.fori_loop(0, _NCH, chunk, 0)


_sc_gather = functools.partial(
    pl.kernel,
    out_type=jax.ShapeDtypeStruct((_N * _K, _C), jnp.float32),
    mesh=plsc.VectorSubcoreMesh(core_axis_name="c", subcore_axis_name="s"),
    scratch_types=[
        pltpu.VMEM((_CH,), jnp.int32),
        pltpu.VMEM((_CH, _C), jnp.float32),
        pltpu.SemaphoreType.DMA,
    ],
    compiler_params=pltpu.CompilerParams(use_tc_tiling_on_sc=False),
)(_sc_gather_body)


_BN = 400
_GRID = _N // _BN


def _tc_body(g_ref, w_ref, w2_ref, b_ref, o_ref):
    g = g_ref[...]
    w = w_ref[...]
    acc = None
    for k in range(_K):
        a = g[:, k * _C:(k + 1) * _C]
        b = w[:, k * _M:(k + 1) * _M]
        ae = jnp.broadcast_to(a[:, :, None], (_BN, _C, _M)).reshape(_BN, _C * _M)
        be = jnp.broadcast_to(b[:, None, :], (_BN, _C, _M)).reshape(_BN, _C * _M)
        t = ae * be
        acc = t if acc is None else acc + t
    out = lax.dot_general(acc, w2_ref[...], (((1,), (0,)), ((), ())),
                          preferred_element_type=jnp.float32)
    o_ref[...] = out + b_ref[0, :]


_tc_call = pl.pallas_call(
    _tc_body,
    out_shape=jax.ShapeDtypeStruct((_N, _OUT), jnp.float32),
    grid=(_GRID,),
    in_specs=[
        pl.BlockSpec((_BN, _K * _C), lambda i: (i, 0)),
        pl.BlockSpec((_BN, _K * _M), lambda i: (i, 0)),
        pl.BlockSpec((_K * _C, _OUT), lambda i: (0, 0)),
        pl.BlockSpec((8, _OUT), lambda i: (0, 0)),
    ],
    out_specs=pl.BlockSpec((_BN, _OUT), lambda i: (i, 0)),
)


def kernel(input_features, neighbor_inds, inverse_neighbors, inverse_k,
           inverse_idx, weightnet, linear_weight, linear_bias):
    table = input_features[0]
    idx = neighbor_inds[0].reshape(_N * _K)
    gathered = _sc_gather(idx, table)
    g2 = gathered.reshape(_N, _K * _C)
    wn = weightnet[0].reshape(_N, _K * _M)
    w2t = linear_weight.T
    bias8 = jnp.tile(linear_bias[None, :], (8, 1))
    out = _tc_call(g2, wn, w2t, bias8)
    return out[None]


# speedup vs baseline: 28.1622x; 9.8030x over previous
import functools

import jax
import jax.numpy as jnp
from jax import lax
from jax.experimental import pallas as pl
from jax.experimental.pallas import tpu as pltpu
from jax.experimental.pallas import tpu_sc as plsc

_B, _N, _K, _C, _M, _OUT = 1, 100000, 16, 16, 16, 64


_NW = 32
_PER_W = (_N * _K) // _NW
_CH = 2000
_NCH = _PER_W // _CH


def _sc_gather_body(idx_hbm, table_hbm, out_hbm, idx_v, rows_v, sem):
    wid = lax.axis_index("s") * 2 + lax.axis_index("c")

    def chunk(j, carry):
        base = wid * _PER_W + j * _CH
        pltpu.sync_copy(idx_hbm.at[pl.ds(base, _CH)], idx_v)
        pltpu.async_copy(table_hbm.at[idx_v], rows_v, sem).wait()
        pltpu.sync_copy(rows_v, out_hbm.at[pl.ds(base, _CH)])
        return carry

    lax.fori_loop(0, _NCH, chunk, 0)


_sc_gather = functools.partial(
    pl.kernel,
    out_type=jax.ShapeDtypeStruct((_N * _K, _C), jnp.float32),
    mesh=plsc.VectorSubcoreMesh(core_axis_name="c", subcore_axis_name="s"),
    scratch_types=[
        pltpu.VMEM((_CH,), jnp.int32),
        pltpu.VMEM((_CH, _C), jnp.float32),
        pltpu.SemaphoreType.DMA,
    ],
    compiler_params=pltpu.CompilerParams(use_tc_tiling_on_sc=False),
)(_sc_gather_body)


_BNL = 128
_GRID = -(-_N // _BNL)


def _tc_body(gt_ref, wt_ref, w2_ref, b_ref, ot_ref):
    accs = [jnp.zeros((_M, _BNL), jnp.float32) for _ in range(_C)]
    for k in range(_K):
        wk = wt_ref[k * _M:(k + 1) * _M, :]
        for c in range(_C):
            grow = gt_ref[k * _C + c, :]
            accs[c] = accs[c] + grow[None, :] * wk
    p = jnp.concatenate(accs, axis=0)
    out_t = lax.dot_general(w2_ref[...], p, (((1,), (0,)), ((), ())),
                            preferred_element_type=jnp.float32)
    ot_ref[...] = out_t + b_ref[...]


_tc_call = pl.pallas_call(
    _tc_body,
    out_shape=jax.ShapeDtypeStruct((_OUT, _N), jnp.float32),
    grid=(_GRID,),
    in_specs=[
        pl.BlockSpec((_K * _C, _BNL), lambda i: (0, i)),
        pl.BlockSpec((_K * _M, _BNL), lambda i: (0, i)),
        pl.BlockSpec((_OUT, _K * _C), lambda i: (0, 0)),
        pl.BlockSpec((_OUT, _BNL), lambda i: (0, 0)),
    ],
    out_specs=pl.BlockSpec((_OUT, _BNL), lambda i: (0, i)),
)


def kernel(input_features, neighbor_inds, inverse_neighbors, inverse_k,
           inverse_idx, weightnet, linear_weight, linear_bias):
    table = input_features[0]
    idx = neighbor_inds[0].reshape(_N * _K)
    gathered = _sc_gather(idx, table)
    g_t = gathered.reshape(_N, _K * _C).T
    w_t = weightnet[0].reshape(_N, _K * _M).T
    bias_t = jnp.tile(linear_bias[:, None], (1, _BNL))
    out_t = _tc_call(g_t, w_t, linear_weight, bias_t)
    return out_t.T[None]


# speedup vs baseline: 35.9688x; 1.2772x over previous
import functools

import jax
import jax.numpy as jnp
from jax import lax
from jax.experimental import pallas as pl
from jax.experimental.pallas import tpu as pltpu
from jax.experimental.pallas import tpu_sc as plsc

_B, _N, _K, _C, _M, _OUT = 1, 100000, 16, 16, 16, 64


_NW = 32
_PER_W = (_N * _K) // _NW
_CH = 2000
_NCH = _PER_W // _CH


def _sc_gather_body(idx_hbm, table_hbm, out_hbm, idx_v, rows_v, sem):
    wid = lax.axis_index("s") * 2 + lax.axis_index("c")

    def chunk(j, carry):
        base = wid * _PER_W + j * _CH
        pltpu.sync_copy(idx_hbm.at[pl.ds(base, _CH)], idx_v)
        pltpu.async_copy(table_hbm.at[idx_v], rows_v, sem).wait()
        pltpu.sync_copy(rows_v, out_hbm.at[pl.ds(base, _CH)])
        return carry

    lax.fori_loop(0, _NCH, chunk, 0)


_sc_gather = functools.partial(
    pl.kernel,
    out_type=jax.ShapeDtypeStruct((_N * _K, _C), jnp.float32),
    mesh=plsc.VectorSubcoreMesh(core_axis_name="c", subcore_axis_name="s"),
    scratch_types=[
        pltpu.VMEM((_CH,), jnp.int32),
        pltpu.VMEM((_CH, _C), jnp.float32),
        pltpu.SemaphoreType.DMA,
    ],
    compiler_params=pltpu.CompilerParams(use_tc_tiling_on_sc=False),
)(_sc_gather_body)


_BNL = 256
_GRID = -(-_N // _BNL)


def _tc_body(gt_ref, wt_ref, w2_ref, b_ref, ot_ref):
    halves = []
    for h in range(2):
        accs = [jnp.zeros((8, _BNL), jnp.float32) for _ in range(_C)]
        for k in range(_K):
            gk = gt_ref[k * _C:(k + 1) * _C, :]
            wkh = wt_ref[k * _M + 8 * h:k * _M + 8 * h + 8, :]
            for c in range(_C):
                accs[c] = accs[c] + gk[c][None, :] * wkh
        halves.append(accs)
    rows = []
    for c in range(_C):
        rows.append(halves[0][c])
        rows.append(halves[1][c])
    p = jnp.concatenate(rows, axis=0)
    out_t = lax.dot_general(w2_ref[...], p, (((1,), (0,)), ((), ())),
                            preferred_element_type=jnp.float32)
    ot_ref[...] = out_t + b_ref[...]


_tc_call = pl.pallas_call(
    _tc_body,
    out_shape=jax.ShapeDtypeStruct((_OUT, _N), jnp.float32),
    grid=(_GRID,),
    in_specs=[
        pl.BlockSpec((_K * _C, _BNL), lambda i: (0, i)),
        pl.BlockSpec((_K * _M, _BNL), lambda i: (0, i)),
        pl.BlockSpec((_OUT, _K * _C), lambda i: (0, 0)),
        pl.BlockSpec((_OUT, _BNL), lambda i: (0, 0)),
    ],
    out_specs=pl.BlockSpec((_OUT, _BNL), lambda i: (0, i)),
)


def kernel(input_features, neighbor_inds, inverse_neighbors, inverse_k,
           inverse_idx, weightnet, linear_weight, linear_bias):
    table = input_features[0]
    idx = neighbor_inds[0].reshape(_N * _K)
    gathered = _sc_gather(idx, table)
    g_t = gathered.reshape(_N, _K * _C).T
    w_t = weightnet[0].reshape(_N, _K * _M).T
    bias_t = jnp.tile(linear_bias[:, None], (1, _BNL))
    out_t = _tc_call(g_t, w_t, linear_weight, bias_t)
    return out_t.T[None]


# speedup vs baseline: 36.2673x; 1.0083x over previous
import functools

import jax
import jax.numpy as jnp
from jax import lax
from jax.experimental import pallas as pl
from jax.experimental.pallas import tpu as pltpu
from jax.experimental.pallas import tpu_sc as plsc

_B, _N, _K, _C, _M, _OUT = 1, 100000, 16, 16, 16, 64


_NW = 32
_PER_W = (_N * _K) // _NW
_CH = 1000
_NCH = _PER_W // _CH
_NPAIR = _NCH // 2


def _sc_gather_body(idx_hbm, table_hbm, out_hbm, idx_v, rows_a, rows_b,
                    gsem, ssem_a, ssem_b):
    wid = lax.axis_index("s") * 2 + lax.axis_index("c")
    pltpu.sync_copy(idx_hbm.at[wid], idx_v)
    base = wid * _PER_W

    def gather(j, rows):
        pltpu.async_copy(table_hbm.at[idx_v.at[j]], rows, gsem).wait()

    def start_store(j, rows, ssem):
        pltpu.make_async_copy(rows, out_hbm.at[pl.ds(base + j * _CH, _CH)],
                              ssem).start()

    def wait_store(rows, ssem):
        pltpu.make_async_copy(rows, out_hbm.at[pl.ds(base, _CH)], ssem).wait()

    gather(0, rows_a)
    start_store(0, rows_a, ssem_a)
    gather(1, rows_b)
    start_store(1, rows_b, ssem_b)

    def pair(i, carry):
        wait_store(rows_a, ssem_a)
        gather(2 * i, rows_a)
        start_store(2 * i, rows_a, ssem_a)
        wait_store(rows_b, ssem_b)
        gather(2 * i + 1, rows_b)
        start_store(2 * i + 1, rows_b, ssem_b)
        return carry

    lax.fori_loop(1, _NPAIR, pair, 0)
    wait_store(rows_a, ssem_a)
    wait_store(rows_b, ssem_b)


_sc_gather = functools.partial(
    pl.kernel,
    out_type=jax.ShapeDtypeStruct((_N * _K, _C), jnp.float32),
    mesh=plsc.VectorSubcoreMesh(core_axis_name="c", subcore_axis_name="s"),
    scratch_types=[
        pltpu.VMEM((_NCH, _CH), jnp.int32),
        pltpu.VMEM((_CH, _C), jnp.float32),
        pltpu.VMEM((_CH, _C), jnp.float32),
        pltpu.SemaphoreType.DMA,
        pltpu.SemaphoreType.DMA,
        pltpu.SemaphoreType.DMA,
    ],
    compiler_params=pltpu.CompilerParams(use_tc_tiling_on_sc=False),
)(_sc_gather_body)


_BNL = 256
_GRID = -(-_N // _BNL)


def _tc_body(gt_ref, wt_ref, w2_ref, b_ref, ot_ref):
    halves = []
    for h in range(2):
        accs = [jnp.zeros((8, _BNL), jnp.float32) for _ in range(_C)]
        for k in range(_K):
            gk = gt_ref[k * _C:(k + 1) * _C, :]
            wkh = wt_ref[k * _M + 8 * h:k * _M + 8 * h + 8, :]
            for c in range(_C):
                accs[c] = accs[c] + gk[c][None, :] * wkh
        halves.append(accs)
    rows = []
    for c in range(_C):
        rows.append(halves[0][c])
        rows.append(halves[1][c])
    p = jnp.concatenate(rows, axis=0)
    out_t = lax.dot_general(w2_ref[...], p, (((1,), (0,)), ((), ())),
                            preferred_element_type=jnp.float32)
    ot_ref[...] = out_t + b_ref[...]


_tc_call = pl.pallas_call(
    _tc_body,
    out_shape=jax.ShapeDtypeStruct((_OUT, _N), jnp.float32),
    grid=(_GRID,),
    in_specs=[
        pl.BlockSpec((_K * _C, _BNL), lambda i: (0, i)),
        pl.BlockSpec((_K * _M, _BNL), lambda i: (0, i)),
        pl.BlockSpec((_OUT, _K * _C), lambda i: (0, 0)),
        pl.BlockSpec((_OUT, _BNL), lambda i: (0, 0)),
    ],
    out_specs=pl.BlockSpec((_OUT, _BNL), lambda i: (0, i)),
)


def kernel(input_features, neighbor_inds, inverse_neighbors, inverse_k,
           inverse_idx, weightnet, linear_weight, linear_bias):
    table = input_features[0]
    idx = neighbor_inds[0].reshape(_NW, _NCH, _CH)
    gathered = _sc_gather(idx, table)
    g_t = gathered.reshape(_N, _K * _C).T
    w_t = weightnet[0].reshape(_N, _K * _M).T
    bias_t = jnp.tile(linear_bias[:, None], (1, _BNL))
    out_t = _tc_call(g_t, w_t, linear_weight, bias_t)
    return out_t.T[None]
